# baseline (speedup 1.0000x reference)
import jax
import jax.numpy as jnp
from jax.experimental import pallas as pl

G, S, D, E, C = 4, 8192, 768, 64, 256
BS = 512


def _gates_body(x_ref, w_ref, out_ref):
    l = jax.lax.dot_general(
        x_ref[...], w_ref[...], (((1,), (0,)), ((), ())),
        precision=jax.lax.Precision.DEFAULT,
        preferred_element_type=jnp.float32)
    m = jnp.max(l, axis=-1, keepdims=True)
    e = jnp.exp(l - m)
    out_ref[...] = e / jnp.sum(e, axis=-1, keepdims=True)


def kernel(inputs, W):
    x = inputs.reshape(G * S, D)
    gates = pl.pallas_call(
        _gates_body,
        grid=(G * S // BS,),
        in_specs=[pl.BlockSpec((BS, D), lambda i: (i, 0)),
                  pl.BlockSpec((D, E), lambda i: (0, 0))],
        out_specs=pl.BlockSpec((BS, E), lambda i: (i, 0)),
        out_shape=jax.ShapeDtypeStruct((G * S, E), jnp.float32),
    )(x, W)
    gates = gates.reshape(G, S, E)

    def _route(g):
        top_gates, top_idx = jax.lax.top_k(g.T, C)
        cnt = jnp.zeros((S,), g.dtype).at[top_idx.reshape(-1)].add(1.0)
        ratio = jnp.mean((cnt > 0).astype(g.dtype))
        return top_gates, top_idx, cnt, ratio

    top_gates, top_idx, cnt, ratio = jax.vmap(_route)(gates)
    aux = jnp.zeros((), gates.dtype)
    return top_gates, top_idx, cnt, ratio, aux


# speedup vs baseline: 3.9354x; 3.9354x over previous
import jax
import jax.numpy as jnp
from jax import lax
from jax.experimental import pallas as pl
from jax.experimental.pallas import tpu as pltpu
from jax.experimental.pallas import tpu_sc as plsc

G, S, D, E, C = 4, 8192, 768, 64, 256
BS = 512
SB = S // BS
NB = 1024
CAP = 272
NV_COL = S // 16


def _gates_body(x_ref, w_ref, out_ref):
    l = lax.dot_general(
        x_ref[...], w_ref[...], (((1,), (0,)), ((), ())),
        precision=lax.Precision.DEFAULT,
        preferred_element_type=jnp.float32)
    m = jnp.max(l, axis=-1, keepdims=True)
    e = jnp.exp(l - m)
    g = e / jnp.sum(e, axis=-1, keepdims=True)
    out_ref[...] = g.T[None]


_mesh = plsc.VectorSubcoreMesh(core_axis_name="c", subcore_axis_name="s")


@pl.kernel(
    out_type=[
        jax.ShapeDtypeStruct((G, E, C), jnp.float32),
        jax.ShapeDtypeStruct((G, E, C), jnp.int32),
        jax.ShapeDtypeStruct((G, 64, 128), jnp.float32),
        jax.ShapeDtypeStruct((G, 128), jnp.float32),
    ],
    mesh=_mesh,
    compiler_params=pltpu.CompilerParams(needs_layout_passes=False),
    scratch_types=[
        pltpu.VMEM((S,), jnp.float32),
        pltpu.VMEM((S,), jnp.int32),
        pltpu.VMEM((S,), jnp.int32),
        pltpu.VMEM((S,), jnp.int32),
        pltpu.VMEM((S,), jnp.int32),
        pltpu.VMEM((CAP,), jnp.int32),
        pltpu.VMEM((CAP,), jnp.int32),
        pltpu.VMEM((CAP,), jnp.int32),
        pltpu.VMEM((CAP,), jnp.int32),
        pltpu.VMEM((NB,), jnp.int32),
        pltpu.VMEM((NB + 16,), jnp.int32),
        pltpu.VMEM((32,), jnp.int32),
        pltpu.VMEM((32,), jnp.int32),
        pltpu.VMEM((C,), jnp.float32),
        pltpu.VMEM((64, 128), jnp.float32),
        pltpu.VMEM((64,), jnp.int32),
        pltpu.VMEM_SHARED((128, 128), jnp.float32),
    ],
)
def _route_kernel(gt, zin, tg, ti, ne, rp,
                  col, bk, bi, ck, ci, rk, ri, sk, si,
                  hist, suf, cnt32, base32, og, lh, idx64, shared):
    c = lax.axis_index("c")
    sid = lax.axis_index("s")
    g2 = sid // 8
    g = c * 2 + g2
    eb = (sid % 8) * 8

    iota16 = lax.iota(jnp.int32, 16)
    zi = jnp.full((16,), 0, jnp.int32)
    onesf = jnp.full((16,), 1.0, jnp.float32)

    pltpu.sync_copy(zin, lh)
    suf[pl.ds(NB, 16)] = zi

    @pl.when(sid % 8 == 0)
    def _():
        pltpu.sync_copy(lh, shared.at[pl.ds(g2 * 64, 64)])

    def mk_idx(k, _):
        idx64[pl.ds(k * 16, 16)] = iota16 + (g2 * 64 + k * 16)
        return 0
    lax.fori_loop(0, 4, mk_idx, 0)
    plsc.subcore_barrier()

    def zero_hist():
        def zh(i, _):
            hist[pl.ds(i * 16, 16)] = zi
            return 0
        lax.fori_loop(0, NB // 16, zh, 0)

    def suffix_boundary(need):
        def sstep(j, carry):
            k = NB // 16 - 1 - j
            h = hist[pl.ds(k * 16, 16)]
            cs = plsc.cumsum(lax.rev(h, (0,))) + carry
            suf[pl.ds(k * 16, 16)] = lax.rev(cs, (0,))
            return jnp.max(cs, axis=0)
        lax.fori_loop(0, NB // 16, sstep, jnp.int32(0))

        def fstep(k, best):
            v = suf[pl.ds(k * 16, 16)]
            cand = jnp.where(v >= need, iota16 + k * 16, -1)
            return jnp.maximum(best, jnp.max(cand, axis=0))
        b = lax.fori_loop(0, NB // 16, fstep, jnp.int32(-1))
        nhi_v = plsc.load_gather(suf, [jnp.full((16,), b + 1, jnp.int32)])
        return b, jnp.max(nhi_v, axis=0)

    def level1():
        zero_hist()

        def ha(i, _):
            k = plsc.bitcast(col[pl.ds(i * 16, 16)], jnp.int32)
            d = lax.shift_right_logical(k, 21) & (NB - 1)
            cnt, last = plsc.scan_count(d)
            plsc.addupdate_scatter(hist, [d], cnt, mask=last)
            return 0
        lax.fori_loop(0, NV_COL, ha, 0)
        b, n_hi = suffix_boundary(jnp.int32(C))

        def cp(i, carry):
            oh, oe = carry
            k = plsc.bitcast(col[pl.ds(i * 16, 16)], jnp.int32)
            ix = iota16 + i * 16
            d = lax.shift_right_logical(k, 21) & (NB - 1)
            mh = d > b
            me = d == b
            plsc.store_compressed(rk.at[pl.ds(oh, 16)], k, mask=mh)
            plsc.store_compressed(ri.at[pl.ds(oh, 16)], ix, mask=mh)
            plsc.store_compressed(bk.at[pl.ds(oe, 16)], k, mask=me)
            plsc.store_compressed(bi.at[pl.ds(oe, 16)], ix, mask=me)
            return (oh + jnp.sum(mh.astype(jnp.int32), axis=0),
                    oe + jnp.sum(me.astype(jnp.int32), axis=0))
        oh, oe = lax.fori_loop(0, NV_COL, cp, (jnp.int32(0), jnp.int32(0)))
        return oe, jnp.int32(C) - n_hi, oh

    def run_level(skr, sir, dkr, dir_, n, need, off, shift):
        nv = (n + 15) // 16
        zero_hist()

        def ha(i, _):
            k = skr[pl.ds(i * 16, 16)]
            d = lax.shift_right_logical(k, shift) & (NB - 1)
            valid = (iota16 + i * 16) < n
            cnt, last = plsc.scan_count(d, mask=valid)
            plsc.addupdate_scatter(hist, [d], cnt,
                                   mask=jnp.logical_and(last, valid))
            return 0
        lax.fori_loop(0, nv, ha, 0)
        b, n_hi = suffix_boundary(need)

        def cp(i, carry):
            oh, oe = carry
            k = skr[pl.ds(i * 16, 16)]
            ix = sir[pl.ds(i * 16, 16)]
            d = lax.shift_right_logical(k, shift) & (NB - 1)
            valid = (iota16 + i * 16) < n
            mh = jnp.logical_and(d > b, valid)
            me = jnp.logical_and(d == b, valid)
            plsc.store_compressed(rk.at[pl.ds(oh, 16)], k, mask=mh)
            plsc.store_compressed(ri.at[pl.ds(oh, 16)], ix, mask=mh)
            plsc.store_compressed(dkr.at[pl.ds(oe, 16)], k, mask=me)
            plsc.store_compressed(dir_.at[pl.ds(oe, 16)], ix, mask=me)
            return (oh + jnp.sum(mh.astype(jnp.int32), axis=0),
                    oe + jnp.sum(me.astype(jnp.int32), axis=0))
        oh, oe = lax.fori_loop(0, nv, cp, (off, jnp.int32(0)))
        return oe, need - n_hi, oh

    def column(j, _):
        e = eb + j
        pltpu.sync_copy(gt.at[g, e], col)

        n_eq, need, off = level1()
        n_eq, need, off = run_level(bk, bi, ck, ci, n_eq, need, off, 11)
        n_eq, need, off = run_level(ck, ci, bk, bi, n_eq, need, off, 1)
        n_eq, need, off = run_level(bk, bi, ck, ci, n_eq, need, off, 0)

        def fc(i, o):
            rk[pl.ds(o, 16)] = ck[pl.ds(i * 16, 16)]
            ri[pl.ds(o, 16)] = ci[pl.ds(i * 16, 16)]
            return o + 16
        lax.fori_loop(0, (need + 15) // 16, fc, off)

        bufs = ((rk, ri), (sk, si))
        for p, shift in enumerate((0, 5, 10, 15, 20, 25, 30)):
            skr, sir = bufs[p % 2]
            dkr, dir_ = bufs[(p + 1) % 2]
            cnt32[pl.ds(0, 16)] = zi
            cnt32[pl.ds(16, 16)] = zi

            def pa(i, _, skr=skr, shift=shift):
                k = skr[pl.ds(i * 16, 16)]
                d = 31 - (lax.shift_right_logical(k, shift) & 31)
                cnt, last = plsc.scan_count(d)
                plsc.addupdate_scatter(cnt32, [d], cnt, mask=last)
                return 0
            lax.fori_loop(0, C // 16, pa, 0)

            v = cnt32[pl.ds(0, 16)]
            w = cnt32[pl.ds(16, 16)]
            cs1 = plsc.cumsum(v)
            cs2 = plsc.cumsum(w)
            base32[pl.ds(0, 16)] = cs1 - v
            base32[pl.ds(16, 16)] = cs2 - w + jnp.max(cs1, axis=0)

            def pb(i, _, skr=skr, sir=sir, dkr=dkr, dir_=dir_, shift=shift):
                k = skr[pl.ds(i * 16, 16)]
                ix = sir[pl.ds(i * 16, 16)]
                d = 31 - (lax.shift_right_logical(k, shift) & 31)
                cnt, last = plsc.scan_count(d)
                bsv = plsc.load_gather(base32, [d])
                pos = bsv + cnt - 1
                plsc.store_scatter(dkr, [pos], k)
                plsc.store_scatter(dir_, [pos], ix)
                plsc.store_scatter(base32, [d], bsv + cnt, mask=last)
                return 0
            lax.fori_loop(0, C // 16, pb, 0)

        def ob(i, _):
            og[pl.ds(i * 16, 16)] = plsc.bitcast(sk[pl.ds(i * 16, 16)],
                                                 jnp.float32)
            iv = si[pl.ds(i * 16, 16)]
            plsc.addupdate_scatter(
                lh, [lax.shift_right_logical(iv, 7), iv & 127], onesf)
            return 0
        lax.fori_loop(0, C // 16, ob, 0)
        pltpu.sync_copy(og, tg.at[g, e])
        pltpu.sync_copy(si.at[pl.ds(0, C)], ti.at[g, e])
        return 0

    lax.fori_loop(0, 8, column, 0)

    pltpu.sync_copy(lh, shared.at[idx64], add=True)
    plsc.subcore_barrier()

    @pl.when(sid % 8 == 0)
    def _():
        pltpu.sync_copy(shared.at[pl.ds(g2 * 64, 64)], lh)
        pltpu.sync_copy(lh, ne.at[g])

        def rstep(r, acc):
            a = acc
            for cseg in range(8):
                x16 = plsc.load_gather(
                    lh, [jnp.full((16,), r, jnp.int32), iota16 + cseg * 16])
                a = a + jnp.where(x16 > 0.0, 1.0, 0.0)
            return a
        accv = lax.fori_loop(0, 64, rstep, jnp.full((16,), 0.0, jnp.float32))
        total = jnp.sum(accv, axis=0)
        rvec = jnp.full((16,), total * (1.0 / S), jnp.float32)

        def wr(k, _):
            og[pl.ds(k * 16, 16)] = rvec
            return 0
        lax.fori_loop(0, 8, wr, 0)
        pltpu.sync_copy(og.at[pl.ds(0, 128)], rp.at[g])


def kernel(inputs, W):
    x = inputs.reshape(G * S, D)
    gates_t = pl.pallas_call(
        _gates_body,
        grid=(G * SB,),
        in_specs=[pl.BlockSpec((BS, D), lambda i: (i, 0)),
                  pl.BlockSpec((D, E), lambda i: (0, 0))],
        out_specs=pl.BlockSpec((1, E, BS), lambda i: (i // SB, 0, i % SB)),
        out_shape=jax.ShapeDtypeStruct((G, E, S), jnp.float32),
    )(x, W)

    zin = jnp.zeros((64, 128), jnp.float32)
    tg, ti, ne, rp = jax.jit(_route_kernel)(gates_t, zin)
    return (tg, ti, ne.reshape(G, S), rp[:, 0],
            jnp.zeros((), jnp.float32))


# speedup vs baseline: 5.5825x; 1.4185x over previous
import jax
import jax.numpy as jnp
from jax import lax
from jax.experimental import pallas as pl
from jax.experimental.pallas import tpu as pltpu
from jax.experimental.pallas import tpu_sc as plsc

G, S, D, E, C = 4, 8192, 768, 64, 256
BS = 512
SB = S // BS
NB = 1024
CAP = 272
NV_COL = S // 16


def _gates_body(x_ref, w_ref, out_ref):
    dn = (((1,), (0,)), ((), ()))
    l = lax.dot_general(
        x_ref[:, :512], w_ref[:512, :], dn,
        precision=lax.Precision.DEFAULT,
        preferred_element_type=jnp.float32)
    l = l + lax.dot_general(
        x_ref[:, 512:], w_ref[512:, :], dn,
        precision=lax.Precision.DEFAULT,
        preferred_element_type=jnp.float32)
    m = jnp.max(l, axis=-1, keepdims=True)
    e = jnp.exp(l - m)
    s = e
    for d in (32, 16, 8, 4, 2, 1):
        s = s[:, :d] + s[:, d:2 * d]
    g = e / s
    out_ref[...] = g.T[None]


_mesh = plsc.VectorSubcoreMesh(core_axis_name="c", subcore_axis_name="s")


@pl.kernel(
    out_type=[
        jax.ShapeDtypeStruct((G, E, C), jnp.float32),
        jax.ShapeDtypeStruct((G, E, C), jnp.int32),
        jax.ShapeDtypeStruct((G, 64, 128), jnp.float32),
        jax.ShapeDtypeStruct((G, 128), jnp.float32),
    ],
    mesh=_mesh,
    compiler_params=pltpu.CompilerParams(needs_layout_passes=False),
    scratch_types=[
        pltpu.VMEM((S,), jnp.float32),
        pltpu.VMEM((S,), jnp.int32),
        pltpu.VMEM((S,), jnp.int32),
        pltpu.VMEM((S,), jnp.int32),
        pltpu.VMEM((S,), jnp.int32),
        pltpu.VMEM((CAP,), jnp.int32),
        pltpu.VMEM((CAP,), jnp.int32),
        pltpu.VMEM((CAP,), jnp.int32),
        pltpu.VMEM((CAP,), jnp.int32),
        pltpu.VMEM((NB,), jnp.int32),
        pltpu.VMEM((NB + 16,), jnp.int32),
        pltpu.VMEM((32,), jnp.int32),
        pltpu.VMEM((32,), jnp.int32),
        pltpu.VMEM((C,), jnp.float32),
        pltpu.VMEM((64, 128), jnp.float32),
        pltpu.VMEM((64,), jnp.int32),
        pltpu.VMEM_SHARED((128, 128), jnp.float32),
    ],
)
def _route_kernel(gt, zin, tg, ti, ne, rp,
                  col, bk, bi, ck, ci, rk, ri, sk, si,
                  hist, suf, cnt32, base32, og, lh, idx64, shared):
    c = lax.axis_index("c")
    sid = lax.axis_index("s")
    g2 = sid // 8
    g = c * 2 + g2
    eb = (sid % 8) * 8

    iota16 = lax.iota(jnp.int32, 16)
    zi = jnp.full((16,), 0, jnp.int32)
    onesf = jnp.full((16,), 1.0, jnp.float32)

    pltpu.sync_copy(zin, lh)
    suf[pl.ds(NB, 16)] = zi

    @pl.when(sid % 8 == 0)
    def _():
        pltpu.sync_copy(lh, shared.at[pl.ds(g2 * 64, 64)])

    def mk_idx(k, _):
        idx64[pl.ds(k * 16, 16)] = iota16 + (g2 * 64 + k * 16)
        return 0
    lax.fori_loop(0, 4, mk_idx, 0)
    plsc.subcore_barrier()

    def zero_hist():
        @plsc.parallel_loop(0, NB // 16, unroll=4)
        def _zh(i):
            hist[pl.ds(i * 16, 16)] = zi

    def suffix_boundary(need):
        def sstep(j, carry):
            k = NB // 16 - 1 - j
            h = hist[pl.ds(k * 16, 16)]
            cs = plsc.cumsum(lax.rev(h, (0,))) + carry
            suf[pl.ds(k * 16, 16)] = lax.rev(cs, (0,))
            return jnp.max(cs, axis=0)
        plsc.parallel_loop(0, NB // 16, unroll=2,
                           carry=jnp.int32(0))(sstep)

        def fstep(k, best):
            v = suf[pl.ds(k * 16, 16)]
            cand = jnp.where(v >= need, iota16 + k * 16, -1)
            return jnp.maximum(best, jnp.max(cand, axis=0))
        b = plsc.parallel_loop(0, NB // 16, unroll=4,
                               carry=jnp.int32(-1))(fstep)
        nhi_v = plsc.load_gather(suf, [jnp.full((16,), b + 1, jnp.int32)])
        return b, jnp.max(nhi_v, axis=0)

    def level1():
        zero_hist()

        @plsc.parallel_loop(0, NV_COL, unroll=4)
        def _ha(i):
            k = plsc.bitcast(col[pl.ds(i * 16, 16)], jnp.int32)
            d = lax.shift_right_logical(k, 21) & (NB - 1)
            cnt, last = plsc.scan_count(d)
            plsc.addupdate_scatter(hist, [d], cnt, mask=last)
        b, n_hi = suffix_boundary(jnp.int32(C))

        def cp(i, carry):
            oh, oe = carry
            k = plsc.bitcast(col[pl.ds(i * 16, 16)], jnp.int32)
            ix = iota16 + i * 16
            d = lax.shift_right_logical(k, 21) & (NB - 1)
            mh = d > b
            me = d == b
            plsc.store_compressed(rk.at[pl.ds(oh, 16)], k, mask=mh)
            plsc.store_compressed(ri.at[pl.ds(oh, 16)], ix, mask=mh)
            plsc.store_compressed(bk.at[pl.ds(oe, 16)], k, mask=me)
            plsc.store_compressed(bi.at[pl.ds(oe, 16)], ix, mask=me)
            return (oh + jnp.sum(mh.astype(jnp.int32), axis=0),
                    oe + jnp.sum(me.astype(jnp.int32), axis=0))
        oh, oe = plsc.parallel_loop(
            0, NV_COL, unroll=4,
            carry=(jnp.int32(0), jnp.int32(0)))(cp)
        return oe, jnp.int32(C) - n_hi, oh

    def run_level(skr, sir, dkr, dir_, n, need, off, shift):
        nv = (n + 15) // 16
        zero_hist()

        def ha(i, _):
            k = skr[pl.ds(i * 16, 16)]
            d = lax.shift_right_logical(k, shift) & (NB - 1)
            valid = (iota16 + i * 16) < n
            cnt, last = plsc.scan_count(d, mask=valid)
            plsc.addupdate_scatter(hist, [d], cnt,
                                   mask=jnp.logical_and(last, valid))
            return 0
        plsc.parallel_loop(0, nv, unroll=2)(lambda i: ha(i, 0))
        b, n_hi = suffix_boundary(need)

        def cp(i, carry):
            oh, oe = carry
            k = skr[pl.ds(i * 16, 16)]
            ix = sir[pl.ds(i * 16, 16)]
            d = lax.shift_right_logical(k, shift) & (NB - 1)
            valid = (iota16 + i * 16) < n
            mh = jnp.logical_and(d > b, valid)
            me = jnp.logical_and(d == b, valid)
            plsc.store_compressed(rk.at[pl.ds(oh, 16)], k, mask=mh)
            plsc.store_compressed(ri.at[pl.ds(oh, 16)], ix, mask=mh)
            plsc.store_compressed(dkr.at[pl.ds(oe, 16)], k, mask=me)
            plsc.store_compressed(dir_.at[pl.ds(oe, 16)], ix, mask=me)
            return (oh + jnp.sum(mh.astype(jnp.int32), axis=0),
                    oe + jnp.sum(me.astype(jnp.int32), axis=0))
        oh, oe = plsc.parallel_loop(
            0, nv, unroll=2, carry=(off, jnp.int32(0)))(cp)
        return oe, need - n_hi, oh

    def column(j, _):
        e = eb + j
        pltpu.sync_copy(gt.at[g, e], col)

        n_eq, need, off = level1()
        n_eq, need, off = run_level(bk, bi, ck, ci, n_eq, need, off, 11)
        n_eq, need, off = run_level(ck, ci, bk, bi, n_eq, need, off, 1)
        n_eq, need, off = run_level(bk, bi, ck, ci, n_eq, need, off, 0)

        def fc(i, o):
            rk[pl.ds(o, 16)] = ck[pl.ds(i * 16, 16)]
            ri[pl.ds(o, 16)] = ci[pl.ds(i * 16, 16)]
            return o + 16
        lax.fori_loop(0, (need + 15) // 16, fc, off)

        bufs = ((rk, ri), (sk, si))
        for p, shift in enumerate((0, 5, 10, 15, 20, 25, 30)):
            skr, sir = bufs[p % 2]
            dkr, dir_ = bufs[(p + 1) % 2]
            cnt32[pl.ds(0, 16)] = zi
            cnt32[pl.ds(16, 16)] = zi

            def pa(i, skr=skr, shift=shift):
                k = skr[pl.ds(i * 16, 16)]
                d = 31 - (lax.shift_right_logical(k, shift) & 31)
                cnt, last = plsc.scan_count(d)
                plsc.addupdate_scatter(cnt32, [d], cnt, mask=last)
            plsc.parallel_loop(0, C // 16, unroll=4)(pa)

            v = cnt32[pl.ds(0, 16)]
            w = cnt32[pl.ds(16, 16)]
            cs1 = plsc.cumsum(v)
            cs2 = plsc.cumsum(w)
            base32[pl.ds(0, 16)] = cs1 - v
            base32[pl.ds(16, 16)] = cs2 - w + jnp.max(cs1, axis=0)

            def pb(i, _, skr=skr, sir=sir, dkr=dkr, dir_=dir_, shift=shift):
                k = skr[pl.ds(i * 16, 16)]
                ix = sir[pl.ds(i * 16, 16)]
                d = 31 - (lax.shift_right_logical(k, shift) & 31)
                cnt, last = plsc.scan_count(d)
                bsv = plsc.load_gather(base32, [d])
                pos = bsv + cnt - 1
                plsc.store_scatter(dkr, [pos], k)
                plsc.store_scatter(dir_, [pos], ix)
                plsc.store_scatter(base32, [d], bsv + cnt, mask=last)
                return 0
            lax.fori_loop(0, C // 16, pb, 0)

        @plsc.parallel_loop(0, C // 16, unroll=4)
        def _ob(i):
            og[pl.ds(i * 16, 16)] = plsc.bitcast(sk[pl.ds(i * 16, 16)],
                                                 jnp.float32)
            iv = si[pl.ds(i * 16, 16)]
            plsc.addupdate_scatter(
                lh, [lax.shift_right_logical(iv, 7), iv & 127], onesf)
        pltpu.sync_copy(og, tg.at[g, e])
        pltpu.sync_copy(si.at[pl.ds(0, C)], ti.at[g, e])
        return 0

    lax.fori_loop(0, 8, column, 0)

    pltpu.sync_copy(lh, shared.at[idx64], add=True)
    plsc.subcore_barrier()

    @pl.when(sid % 8 == 0)
    def _():
        pltpu.sync_copy(shared.at[pl.ds(g2 * 64, 64)], lh)
        pltpu.sync_copy(lh, ne.at[g])

        def rstep(r, acc):
            a = acc
            for cseg in range(8):
                x16 = plsc.load_gather(
                    lh, [jnp.full((16,), r, jnp.int32), iota16 + cseg * 16])
                a = a + jnp.where(x16 > 0.0, 1.0, 0.0)
            return a
        accv = lax.fori_loop(0, 64, rstep, jnp.full((16,), 0.0, jnp.float32))
        total = jnp.sum(accv, axis=0)
        rvec = jnp.full((16,), total * (1.0 / S), jnp.float32)

        def wr(k, _):
            og[pl.ds(k * 16, 16)] = rvec
            return 0
        lax.fori_loop(0, 8, wr, 0)
        pltpu.sync_copy(og.at[pl.ds(0, 128)], rp.at[g])


def kernel(inputs, W):
    x = inputs.reshape(G * S, D)
    gates_t = pl.pallas_call(
        _gates_body,
        grid=(G * SB,),
        in_specs=[pl.BlockSpec((BS, D), lambda i: (i, 0)),
                  pl.BlockSpec((D, E), lambda i: (0, 0))],
        out_specs=pl.BlockSpec((1, E, BS), lambda i: (i // SB, 0, i % SB)),
        out_shape=jax.ShapeDtypeStruct((G, E, S), jnp.float32),
    )(x, W)

    zin = jnp.zeros((64, 128), jnp.float32)
    tg, ti, ne, rp = jax.jit(_route_kernel)(gates_t, zin)
    return (tg, ti, ne.reshape(G, S), rp[:, 0],
            jnp.zeros((), jnp.float32))
